# Initial kernel scaffold; baseline (speedup 1.0000x reference)
import jax
import jax.numpy as jnp
from jax.experimental import pallas as pl


def kernel(x, edge_index, W1, b1, W2, b2):
    raise NotImplementedError("write your pallas kernel here")


# speedup vs baseline: 13.9162x; 13.9162x over previous
import functools

import jax
import jax.numpy as jnp
from jax import lax
from jax.experimental import pallas as pl
from jax.experimental.pallas import tpu as pltpu
from jax.experimental.pallas import tpu_sc as plsc

N = 10000
E = 320000
K = 10
ALPHA = 0.1

NTILES = 16
NCORES = 2
EPT = E // NTILES
CH = 128
NFULL = EPT // CH
TAIL = EPT - NFULL * CH
NCHUNK = NFULL + 1
RPT = 640
NPAD = RPT * NTILES
DUMMY = N
RC = 128
NRC = RPT // RC
C = 32
NV = C // 16


def _mm_kernel(x_ref, w_ref, b_ref, o_ref, *, relu):
  x = x_ref[...]
  if relu:
    x = jnp.maximum(x, 0.0)
  o_ref[...] = jnp.dot(x, w_ref[...], preferred_element_type=jnp.float32) + b_ref[...]


def _matmul(x, w, b, relu=False):
  n, d_in = x.shape
  d_out = w.shape[1]
  blk = 1000
  grid = n // blk
  return pl.pallas_call(
      functools.partial(_mm_kernel, relu=relu),
      grid=(grid,),
      in_specs=[
          pl.BlockSpec((blk, d_in), lambda i: (i, 0)),
          pl.BlockSpec((d_in, d_out), lambda i: (0, 0)),
          pl.BlockSpec((d_out,), lambda i: (0,)),
      ],
      out_specs=pl.BlockSpec((blk, d_out), lambda i: (i, 0)),
      out_shape=jax.ShapeDtypeStruct((n, d_out), jnp.float32),
  )(x, w, b)


def _lane_splat(vec, rr):
  return lax.gather(
      vec, jnp.full((16, 1), rr, jnp.int32),
      dimension_numbers=lax.GatherDimensionNumbers(
          offset_dims=(), collapsed_slice_dims=(0,), start_index_map=(0,)),
      slice_sizes=(1,),
      mode=lax.GatherScatterMode.PROMISE_IN_BOUNDS)


def _appnp_body(NBLK, h_hbm, src_hbm, dst_hbm, out_hbm,
                src2d, dst2d, rowbuf, ubuf, g_tile,
                degb, dinvb, d2b, sdegb, onesb,
                w_sp, acc_sp, deg_sp, sem):
  cid = lax.axis_index("c")
  sid = lax.axis_index("s")
  ebase = sid * EPT
  rbase = sid * RPT

  def load_idx(ci, _):
    pltpu.sync_copy(src_hbm.at[pl.ds(ebase + ci * CH, CH)], src2d.at[ci])
    pltpu.sync_copy(dst_hbm.at[pl.ds(ebase + ci * CH, CH)], dst2d.at[ci])
    return 0
  lax.fori_loop(0, NFULL, load_idx, 0)
  pltpu.sync_copy(src_hbm.at[pl.ds(ebase + NFULL * CH, TAIL)],
                  src2d.at[NFULL, pl.ds(0, TAIL)])
  pltpu.sync_copy(dst_hbm.at[pl.ds(ebase + NFULL * CH, TAIL)],
                  dst2d.at[NFULL, pl.ds(0, TAIL)])
  for j in range(TAIL // 16, CH // 16):
    src2d[NFULL, pl.ds(j * 16, 16)] = jnp.zeros((16,), jnp.int32)
    dst2d[NFULL, pl.ds(j * 16, 16)] = jnp.full((16,), DUMMY, jnp.int32)

  for j in range(RPT // 16):
    degb[pl.ds(j * 16, 16)] = jnp.zeros((16,), jnp.float32)
  for j in range(CH // 16):
    onesb[pl.ds(j * 16, 16)] = jnp.ones((16,), jnp.float32)
  pltpu.sync_copy(degb, deg_sp.at[pl.ds(rbase, RPT)])
  plsc.subcore_barrier()

  def deg_step(ci, _):
    pltpu.sync_copy(onesb, deg_sp.at[dst2d.at[ci]], add=True)
    return 0
  lax.fori_loop(0, NCHUNK, deg_step, 0)
  plsc.subcore_barrier()

  pltpu.sync_copy(deg_sp.at[pl.ds(rbase, RPT)], degb)
  for j in range(RPT // 16):
    sl = pl.ds(j * 16, 16)
    x = degb[sl] + 1.0
    i32 = lax.bitcast_convert_type(x, jnp.int32)
    i32 = jnp.full((16,), 0x5F3759DF, jnp.int32) - lax.shift_right_logical(
        i32, jnp.full((16,), 1, jnp.int32))
    y = lax.bitcast_convert_type(i32, jnp.float32)
    for _ in range(3):
      y = y * (1.5 - 0.5 * x * y * y)
    dinvb[sl] = y
    d2b[sl] = y * y
    sdegb[sl] = 1.0 / y

  for blk in range(NBLK):
    b = cid * NBLK + blk

    for rc in range(NRC):
      for rg in range(RC // 16):
        r0 = rbase + rc * RC + rg * 16

        @pl.when(r0 < N)
        def _():
          pltpu.sync_copy(h_hbm.at[b, pl.ds(r0, 16), pl.ds(0, C)],
                          g_tile.at[pl.ds(rc * RC + rg * 16, 16)])

        @pl.when(r0 >= N)
        def _():
          for rr in range(16):
            for j in range(NV):
              g_tile[rc * RC + rg * 16 + rr, pl.ds(j * 16, 16)] = (
                  jnp.zeros((16,), jnp.float32))

    def grow(rg, _):
      dv = dinvb[pl.ds(rg * 16, 16)]
      for rr in range(16):
        s = _lane_splat(dv, rr)
        r = rg * 16 + rr
        for j in range(NV):
          g_tile[r, pl.ds(j * 16, 16)] = g_tile[r, pl.ds(j * 16, 16)] * s
      return 0
    lax.fori_loop(0, RPT // 16, grow, 0)
    pltpu.sync_copy(g_tile, w_sp.at[pl.ds(rbase, RPT)])
    pltpu.sync_copy(g_tile, acc_sp.at[pl.ds(rbase, RPT)])
    plsc.subcore_barrier()

    def step(k, _):
      final = k == K - 1

      def edge_chunk(ci, _):
        pltpu.async_copy(w_sp.at[src2d.at[ci]], rowbuf, sem).wait()
        pltpu.sync_copy(rowbuf, acc_sp.at[dst2d.at[ci]], add=True)
        return 0
      lax.fori_loop(0, NCHUNK, edge_chunk, 0)
      plsc.subcore_barrier()

      for rc in range(NRC):
        pltpu.sync_copy(acc_sp.at[pl.ds(rbase + rc * RC, RC)], ubuf)

        def urow(rg, _):
          d2v = d2b[pl.ds(rc * RC + rg * 16, 16)]
          sdv = sdegb[pl.ds(rc * RC + rg * 16, 16)]
          for rr in range(16):
            d2s = _lane_splat(d2v, rr)
            scl = jnp.where(final, _lane_splat(sdv, rr),
                            jnp.ones((16,), jnp.float32))
            r = rg * 16 + rr
            for j in range(NV):
              sl = pl.ds(j * 16, 16)
              v = ((1.0 - ALPHA) * d2s * ubuf[r, sl]
                   + ALPHA * g_tile[rc * RC + r, sl])
              ubuf[r, sl] = v * scl
          return 0
        lax.fori_loop(0, RC // 16, urow, 0)

        @pl.when(jnp.logical_not(final))
        def _():
          pltpu.sync_copy(ubuf, w_sp.at[pl.ds(rbase + rc * RC, RC)])
          pltpu.sync_copy(ubuf, acc_sp.at[pl.ds(rbase + rc * RC, RC)])

        @pl.when(final)
        def _():
          for rg in range(RC // 16):
            r0 = rbase + rc * RC + rg * 16

            @pl.when(r0 < N)
            def _():
              pltpu.sync_copy(ubuf.at[pl.ds(rg * 16, 16)],
                              out_hbm.at[b, pl.ds(r0, 16), pl.ds(0, C)])
      plsc.subcore_barrier()
      return 0

    lax.fori_loop(0, K, step, 0)


def _appnp(h, src, dst, d_feat):
  nblk = d_feat // (NCORES * C)
  hsplit = h.reshape(N, NCORES * nblk, C).transpose(1, 0, 2)
  mesh = plsc.VectorSubcoreMesh(core_axis_name="c", subcore_axis_name="s")
  kern = pl.kernel(
      functools.partial(_appnp_body, nblk),
      out_type=jax.ShapeDtypeStruct((NCORES * nblk, N, C), jnp.float32),
      mesh=mesh,
      compiler_params=pltpu.CompilerParams(use_tc_tiling_on_sc=False),
      scratch_types=[
          pltpu.VMEM((NCHUNK, CH), jnp.int32),
          pltpu.VMEM((NCHUNK, CH), jnp.int32),
          pltpu.VMEM((CH, C), jnp.float32),
          pltpu.VMEM((RC, C), jnp.float32),
          pltpu.VMEM((RPT, C), jnp.float32),
          pltpu.VMEM((RPT,), jnp.float32),
          pltpu.VMEM((RPT,), jnp.float32),
          pltpu.VMEM((RPT,), jnp.float32),
          pltpu.VMEM((RPT,), jnp.float32),
          pltpu.VMEM((CH,), jnp.float32),
          pltpu.VMEM_SHARED((NPAD, C), jnp.float32),
          pltpu.VMEM_SHARED((NPAD, C), jnp.float32),
          pltpu.VMEM_SHARED((NPAD,), jnp.float32),
          pltpu.SemaphoreType.DMA,
      ],
  )
  out = kern(hsplit, src, dst)
  return out.transpose(1, 0, 2).reshape(N, d_feat)


def kernel(x, edge_index, W1, b1, W2, b2):
  src = edge_index[0]
  dst = edge_index[1]
  h = _matmul(x, W1, b1)
  z1 = _appnp(h, src, dst, 128)
  h2 = _matmul(z1, W2, b2, relu=True)
  return _appnp(h2, src, dst, 64)


# speedup vs baseline: 20.9911x; 1.5084x over previous
import functools

import jax
import jax.numpy as jnp
from jax import lax
from jax.experimental import pallas as pl
from jax.experimental.pallas import tpu as pltpu
from jax.experimental.pallas import tpu_sc as plsc

N = 10000
E = 320000
K = 10
ALPHA = 0.1

NTILES = 16
NCORES = 2
EPT = E // NTILES
CH = 128
NFULL = EPT // CH
TAIL = EPT - NFULL * CH
NCHUNK = 160
RPT = 640
NPAD = RPT * NTILES
DUMMY = N
RC = 128
NRC = RPT // RC
C = 32
NV = C // 16
NBUF = 4
LEAD = NBUF - 2
NT = NCHUNK // NBUF


def _mm_kernel(x_ref, w_ref, b_ref, o_ref, *, relu):
  x = x_ref[...]
  if relu:
    x = jnp.maximum(x, 0.0)
  o_ref[...] = jnp.dot(x, w_ref[...], preferred_element_type=jnp.float32) + b_ref[...]


def _matmul(x, w, b, relu=False):
  n, d_in = x.shape
  d_out = w.shape[1]
  blk = 1000
  grid = n // blk
  return pl.pallas_call(
      functools.partial(_mm_kernel, relu=relu),
      grid=(grid,),
      in_specs=[
          pl.BlockSpec((blk, d_in), lambda i: (i, 0)),
          pl.BlockSpec((d_in, d_out), lambda i: (0, 0)),
          pl.BlockSpec((d_out,), lambda i: (0,)),
      ],
      out_specs=pl.BlockSpec((blk, d_out), lambda i: (i, 0)),
      out_shape=jax.ShapeDtypeStruct((n, d_out), jnp.float32),
  )(x, w, b)


def _lane_splat(vec, rr):
  return lax.gather(
      vec, jnp.full((16, 1), rr, jnp.int32),
      dimension_numbers=lax.GatherDimensionNumbers(
          offset_dims=(), collapsed_slice_dims=(0,), start_index_map=(0,)),
      slice_sizes=(1,),
      mode=lax.GatherScatterMode.PROMISE_IN_BOUNDS)


def _appnp_body(NBLK, h_hbm, src_hbm, dst_hbm, out_hbm,
                src2d, dst2d, rowbufs, ubuf, g_tile,
                degb, dinvb, d2b, sdegb, onesb,
                w_sp, acc_sp, deg_sp, gsems, ssems, lsem):
  cid = lax.axis_index("c")
  sid = lax.axis_index("s")
  ebase = sid * EPT
  rbase = sid * RPT

  def fire_gather(c, b):
    return pltpu.async_copy(w_sp.at[src2d.at[c]], rowbufs.at[b], gsems.at[b])

  def wait_gather(b):
    pltpu.make_async_copy(w_sp.at[src2d.at[0]], rowbufs.at[b],
                          gsems.at[b]).wait()

  def fire_scatter(c, b):
    return pltpu.async_copy(rowbufs.at[b], acc_sp.at[dst2d.at[c]],
                            ssems.at[b], add=True)

  def wait_scatter(b):
    pltpu.make_async_copy(rowbufs.at[b], acc_sp.at[dst2d.at[0]],
                          ssems.at[b]).wait()

  def fire_load(ci, _):
    pltpu.async_copy(src_hbm.at[pl.ds(ebase + ci * CH, CH)], src2d.at[ci], lsem)
    pltpu.async_copy(dst_hbm.at[pl.ds(ebase + ci * CH, CH)], dst2d.at[ci], lsem)
    return 0
  lax.fori_loop(0, NFULL, fire_load, 0)
  pltpu.async_copy(src_hbm.at[pl.ds(ebase + NFULL * CH, TAIL)],
                   src2d.at[NFULL, pl.ds(0, TAIL)], lsem)
  pltpu.async_copy(dst_hbm.at[pl.ds(ebase + NFULL * CH, TAIL)],
                   dst2d.at[NFULL, pl.ds(0, TAIL)], lsem)

  def drain_load(ci, _):
    pltpu.make_async_copy(src_hbm.at[pl.ds(ebase, CH)], src2d.at[0], lsem).wait()
    pltpu.make_async_copy(dst_hbm.at[pl.ds(ebase, CH)], dst2d.at[0], lsem).wait()
    return 0
  lax.fori_loop(0, NFULL, drain_load, 0)
  pltpu.make_async_copy(src_hbm.at[pl.ds(ebase, TAIL)],
                        src2d.at[0, pl.ds(0, TAIL)], lsem).wait()
  pltpu.make_async_copy(dst_hbm.at[pl.ds(ebase, TAIL)],
                        dst2d.at[0, pl.ds(0, TAIL)], lsem).wait()

  for j in range(TAIL // 16, CH // 16):
    src2d[NFULL, pl.ds(j * 16, 16)] = jnp.zeros((16,), jnp.int32)
    dst2d[NFULL, pl.ds(j * 16, 16)] = jnp.full((16,), DUMMY, jnp.int32)
  for ci in range(NFULL + 1, NCHUNK):
    for j in range(CH // 16):
      src2d[ci, pl.ds(j * 16, 16)] = jnp.zeros((16,), jnp.int32)
      dst2d[ci, pl.ds(j * 16, 16)] = jnp.full((16,), DUMMY, jnp.int32)

  for j in range(RPT // 16):
    degb[pl.ds(j * 16, 16)] = jnp.zeros((16,), jnp.float32)
  for j in range(CH // 16):
    onesb[pl.ds(j * 16, 16)] = jnp.ones((16,), jnp.float32)
  pltpu.sync_copy(degb, deg_sp.at[pl.ds(rbase, RPT)])
  plsc.subcore_barrier()

  def deg_step(ci, _):
    pltpu.sync_copy(onesb, deg_sp.at[dst2d.at[ci]], add=True)
    return 0
  lax.fori_loop(0, NCHUNK, deg_step, 0)
  plsc.subcore_barrier()

  pltpu.sync_copy(deg_sp.at[pl.ds(rbase, RPT)], degb)
  for j in range(RPT // 16):
    sl = pl.ds(j * 16, 16)
    x = degb[sl] + 1.0
    i32 = lax.bitcast_convert_type(x, jnp.int32)
    i32 = jnp.full((16,), 0x5F3759DF, jnp.int32) - lax.shift_right_logical(
        i32, jnp.full((16,), 1, jnp.int32))
    y = lax.bitcast_convert_type(i32, jnp.float32)
    for _ in range(3):
      y = y * (1.5 - 0.5 * x * y * y)
    dinvb[sl] = y
    d2b[sl] = y * y
    sdegb[sl] = 1.0 / y

  for blk in range(NBLK):
    bidx = cid * NBLK + blk

    for rc in range(NRC):
      for rg in range(RC // 16):
        r0 = rbase + rc * RC + rg * 16

        @pl.when(r0 < N)
        def _():
          pltpu.sync_copy(h_hbm.at[bidx, pl.ds(r0, 16), pl.ds(0, C)],
                          g_tile.at[pl.ds(rc * RC + rg * 16, 16)])

        @pl.when(r0 >= N)
        def _():
          for rr in range(16):
            for j in range(NV):
              g_tile[rc * RC + rg * 16 + rr, pl.ds(j * 16, 16)] = (
                  jnp.zeros((16,), jnp.float32))

    def grow(rg, _):
      dv = dinvb[pl.ds(rg * 16, 16)]
      for rr in range(16):
        s = _lane_splat(dv, rr)
        r = rg * 16 + rr
        for j in range(NV):
          g_tile[r, pl.ds(j * 16, 16)] = g_tile[r, pl.ds(j * 16, 16)] * s
      return 0
    lax.fori_loop(0, RPT // 16, grow, 0)
    pltpu.sync_copy(g_tile, w_sp.at[pl.ds(rbase, RPT)])
    pltpu.sync_copy(g_tile, acc_sp.at[pl.ds(rbase, RPT)])
    plsc.subcore_barrier()

    def step(k, _):
      final = k == K - 1

      for b in range(LEAD):
        fire_gather(jnp.int32(b), b)

      def edge_iter(t, _):
        for b in range(NBUF):
          c = t * NBUF + b
          wait_gather(b)
          fire_scatter(c, b)
          g = c + LEAD
          bg = (b + LEAD) % NBUF
          if b < NBUF - LEAD:
            @pl.when(t >= 1)
            def _():
              wait_scatter(bg)
            fire_gather(g, bg)
          else:
            @pl.when(t < NT - 1)
            def _():
              wait_scatter(bg)
              fire_gather(g, bg)
        return 0
      lax.fori_loop(0, NT, edge_iter, 0)
      for b in range(NBUF):
        wait_scatter(b)
      plsc.subcore_barrier()

      for rc in range(NRC):
        pltpu.sync_copy(acc_sp.at[pl.ds(rbase + rc * RC, RC)], ubuf)

        def urow(rg, _):
          d2v = d2b[pl.ds(rc * RC + rg * 16, 16)]
          sdv = sdegb[pl.ds(rc * RC + rg * 16, 16)]
          for rr in range(16):
            d2s = _lane_splat(d2v, rr)
            scl = jnp.where(final, _lane_splat(sdv, rr),
                            jnp.ones((16,), jnp.float32))
            r = rg * 16 + rr
            for j in range(NV):
              sl = pl.ds(j * 16, 16)
              v = ((1.0 - ALPHA) * d2s * ubuf[r, sl]
                   + ALPHA * g_tile[rc * RC + r, sl])
              ubuf[r, sl] = v * scl
          return 0
        lax.fori_loop(0, RC // 16, urow, 0)

        @pl.when(jnp.logical_not(final))
        def _():
          pltpu.sync_copy(ubuf, w_sp.at[pl.ds(rbase + rc * RC, RC)])
          pltpu.sync_copy(ubuf, acc_sp.at[pl.ds(rbase + rc * RC, RC)])

        @pl.when(final)
        def _():
          for rg in range(RC // 16):
            r0 = rbase + rc * RC + rg * 16

            @pl.when(r0 < N)
            def _():
              pltpu.sync_copy(ubuf.at[pl.ds(rg * 16, 16)],
                              out_hbm.at[bidx, pl.ds(r0, 16), pl.ds(0, C)])
      plsc.subcore_barrier()
      return 0

    lax.fori_loop(0, K, step, 0)


def _appnp(h, src, dst, d_feat):
  nblk = d_feat // (NCORES * C)
  hsplit = h.reshape(N, NCORES * nblk, C).transpose(1, 0, 2)
  mesh = plsc.VectorSubcoreMesh(core_axis_name="c", subcore_axis_name="s")
  kern = pl.kernel(
      functools.partial(_appnp_body, nblk),
      out_type=jax.ShapeDtypeStruct((NCORES * nblk, N, C), jnp.float32),
      mesh=mesh,
      compiler_params=pltpu.CompilerParams(use_tc_tiling_on_sc=False),
      scratch_types=[
          pltpu.VMEM((NCHUNK, CH), jnp.int32),
          pltpu.VMEM((NCHUNK, CH), jnp.int32),
          pltpu.VMEM((NBUF, CH, C), jnp.float32),
          pltpu.VMEM((RC, C), jnp.float32),
          pltpu.VMEM((RPT, C), jnp.float32),
          pltpu.VMEM((RPT,), jnp.float32),
          pltpu.VMEM((RPT,), jnp.float32),
          pltpu.VMEM((RPT,), jnp.float32),
          pltpu.VMEM((RPT,), jnp.float32),
          pltpu.VMEM((CH,), jnp.float32),
          pltpu.VMEM_SHARED((NPAD, C), jnp.float32),
          pltpu.VMEM_SHARED((NPAD, C), jnp.float32),
          pltpu.VMEM_SHARED((NPAD,), jnp.float32),
          pltpu.SemaphoreType.DMA((NBUF,)),
          pltpu.SemaphoreType.DMA((NBUF,)),
          pltpu.SemaphoreType.DMA,
      ],
  )
  out = kern(hsplit, src, dst)
  return out.transpose(1, 0, 2).reshape(N, d_feat)


def kernel(x, edge_index, W1, b1, W2, b2):
  src = edge_index[0]
  dst = edge_index[1]
  h = _matmul(x, W1, b1)
  z1 = _appnp(h, src, dst, 128)
  h2 = _matmul(z1, W2, b2, relu=True)
  return _appnp(h2, src, dst, 64)


# speedup vs baseline: 21.0690x; 1.0037x over previous
import functools

import jax
import jax.numpy as jnp
from jax import lax
from jax.experimental import pallas as pl
from jax.experimental.pallas import tpu as pltpu
from jax.experimental.pallas import tpu_sc as plsc

N = 10000
E = 320000
K = 10
ALPHA = 0.1

NTILES = 16
NCORES = 2
EPT = E // NTILES
CH = 128
NFULL = EPT // CH
TAIL = EPT - NFULL * CH
NCHUNK = 160
RPT = 640
NPAD = RPT * NTILES
DUMMY = N
RC = 128
NRC = RPT // RC
C = 32
NV = C // 16
NBUF = 5
LEAD = NBUF - 2
NT = NCHUNK // NBUF


def _mm_kernel(x_ref, w_ref, b_ref, o_ref, *, relu):
  x = x_ref[...]
  if relu:
    x = jnp.maximum(x, 0.0)
  o_ref[...] = jnp.dot(x, w_ref[...], preferred_element_type=jnp.float32) + b_ref[...]


def _matmul(x, w, b, relu=False):
  n, d_in = x.shape
  d_out = w.shape[1]
  blk = 1000
  grid = n // blk
  return pl.pallas_call(
      functools.partial(_mm_kernel, relu=relu),
      grid=(grid,),
      in_specs=[
          pl.BlockSpec((blk, d_in), lambda i: (i, 0)),
          pl.BlockSpec((d_in, d_out), lambda i: (0, 0)),
          pl.BlockSpec((d_out,), lambda i: (0,)),
      ],
      out_specs=pl.BlockSpec((blk, d_out), lambda i: (i, 0)),
      out_shape=jax.ShapeDtypeStruct((n, d_out), jnp.float32),
  )(x, w, b)


def _lane_splat(vec, rr):
  return lax.gather(
      vec, jnp.full((16, 1), rr, jnp.int32),
      dimension_numbers=lax.GatherDimensionNumbers(
          offset_dims=(), collapsed_slice_dims=(0,), start_index_map=(0,)),
      slice_sizes=(1,),
      mode=lax.GatherScatterMode.PROMISE_IN_BOUNDS)


def _appnp_body(NBLK, h_hbm, src_hbm, dst_hbm, out_hbm,
                src2d, dst2d, rowbufs, ubuf, g_tile,
                degb, dinvb, d2b, sdegb, onesb,
                w_sp, acc_sp, deg_sp, gsems, ssems, lsem):
  cid = lax.axis_index("c")
  sid = lax.axis_index("s")
  ebase = sid * EPT
  rbase = sid * RPT

  def fire_gather(c, b):
    return pltpu.async_copy(w_sp.at[src2d.at[c]], rowbufs.at[b], gsems.at[b])

  def wait_gather(b):
    pltpu.make_async_copy(w_sp.at[src2d.at[0]], rowbufs.at[b],
                          gsems.at[b]).wait()

  def fire_scatter(c, b):
    return pltpu.async_copy(rowbufs.at[b], acc_sp.at[dst2d.at[c]],
                            ssems.at[b], add=True)

  def wait_scatter(b):
    pltpu.make_async_copy(rowbufs.at[b], acc_sp.at[dst2d.at[0]],
                          ssems.at[b]).wait()

  def fire_load(ci, _):
    pltpu.async_copy(src_hbm.at[pl.ds(ebase + ci * CH, CH)], src2d.at[ci], lsem)
    pltpu.async_copy(dst_hbm.at[pl.ds(ebase + ci * CH, CH)], dst2d.at[ci], lsem)
    return 0
  lax.fori_loop(0, NFULL, fire_load, 0)
  pltpu.async_copy(src_hbm.at[pl.ds(ebase + NFULL * CH, TAIL)],
                   src2d.at[NFULL, pl.ds(0, TAIL)], lsem)
  pltpu.async_copy(dst_hbm.at[pl.ds(ebase + NFULL * CH, TAIL)],
                   dst2d.at[NFULL, pl.ds(0, TAIL)], lsem)

  def drain_load(ci, _):
    pltpu.make_async_copy(src_hbm.at[pl.ds(ebase, CH)], src2d.at[0], lsem).wait()
    pltpu.make_async_copy(dst_hbm.at[pl.ds(ebase, CH)], dst2d.at[0], lsem).wait()
    return 0
  lax.fori_loop(0, NFULL, drain_load, 0)
  pltpu.make_async_copy(src_hbm.at[pl.ds(ebase, TAIL)],
                        src2d.at[0, pl.ds(0, TAIL)], lsem).wait()
  pltpu.make_async_copy(dst_hbm.at[pl.ds(ebase, TAIL)],
                        dst2d.at[0, pl.ds(0, TAIL)], lsem).wait()

  for j in range(TAIL // 16, CH // 16):
    src2d[NFULL, pl.ds(j * 16, 16)] = jnp.zeros((16,), jnp.int32)
    dst2d[NFULL, pl.ds(j * 16, 16)] = jnp.full((16,), DUMMY, jnp.int32)
  for ci in range(NFULL + 1, NCHUNK):
    for j in range(CH // 16):
      src2d[ci, pl.ds(j * 16, 16)] = jnp.zeros((16,), jnp.int32)
      dst2d[ci, pl.ds(j * 16, 16)] = jnp.full((16,), DUMMY, jnp.int32)

  for j in range(RPT // 16):
    degb[pl.ds(j * 16, 16)] = jnp.zeros((16,), jnp.float32)
  for j in range(CH // 16):
    onesb[pl.ds(j * 16, 16)] = jnp.ones((16,), jnp.float32)
  pltpu.sync_copy(degb, deg_sp.at[pl.ds(rbase, RPT)])
  plsc.subcore_barrier()

  def deg_step(ci, _):
    pltpu.sync_copy(onesb, deg_sp.at[dst2d.at[ci]], add=True)
    return 0
  lax.fori_loop(0, NCHUNK, deg_step, 0)
  plsc.subcore_barrier()

  pltpu.sync_copy(deg_sp.at[pl.ds(rbase, RPT)], degb)
  for j in range(RPT // 16):
    sl = pl.ds(j * 16, 16)
    x = degb[sl] + 1.0
    i32 = lax.bitcast_convert_type(x, jnp.int32)
    i32 = jnp.full((16,), 0x5F3759DF, jnp.int32) - lax.shift_right_logical(
        i32, jnp.full((16,), 1, jnp.int32))
    y = lax.bitcast_convert_type(i32, jnp.float32)
    for _ in range(3):
      y = y * (1.5 - 0.5 * x * y * y)
    dinvb[sl] = y
    d2b[sl] = y * y
    sdegb[sl] = 1.0 / y

  for blk in range(NBLK):
    bidx = cid * NBLK + blk

    for rc in range(NRC):
      for rg in range(RC // 16):
        r0 = rbase + rc * RC + rg * 16

        @pl.when(r0 < N)
        def _():
          pltpu.sync_copy(h_hbm.at[bidx, pl.ds(r0, 16), pl.ds(0, C)],
                          g_tile.at[pl.ds(rc * RC + rg * 16, 16)])

        @pl.when(r0 >= N)
        def _():
          for rr in range(16):
            for j in range(NV):
              g_tile[rc * RC + rg * 16 + rr, pl.ds(j * 16, 16)] = (
                  jnp.zeros((16,), jnp.float32))

    def grow(rg, _):
      dv = dinvb[pl.ds(rg * 16, 16)]
      for rr in range(16):
        s = _lane_splat(dv, rr)
        r = rg * 16 + rr
        for j in range(NV):
          g_tile[r, pl.ds(j * 16, 16)] = g_tile[r, pl.ds(j * 16, 16)] * s
      return 0
    lax.fori_loop(0, RPT // 16, grow, 0)
    pltpu.sync_copy(g_tile, w_sp.at[pl.ds(rbase, RPT)])
    pltpu.sync_copy(g_tile, acc_sp.at[pl.ds(rbase, RPT)])
    plsc.subcore_barrier()

    def step(k, _):
      final = k == K - 1

      for b in range(LEAD):
        fire_gather(jnp.int32(b), b)

      def edge_iter(t, _):
        for b in range(NBUF):
          c = t * NBUF + b
          wait_gather(b)
          fire_scatter(c, b)
          g = c + LEAD
          bg = (b + LEAD) % NBUF
          if b < NBUF - LEAD:
            @pl.when(t >= 1)
            def _():
              wait_scatter(bg)
            fire_gather(g, bg)
          else:
            @pl.when(t < NT - 1)
            def _():
              wait_scatter(bg)
              fire_gather(g, bg)
        return 0
      lax.fori_loop(0, NT, edge_iter, 0)
      for b in range(NBUF):
        wait_scatter(b)
      plsc.subcore_barrier()

      for rc in range(NRC):
        pltpu.sync_copy(acc_sp.at[pl.ds(rbase + rc * RC, RC)], ubuf)

        def urow(rg, _):
          d2v = d2b[pl.ds(rc * RC + rg * 16, 16)]
          sdv = sdegb[pl.ds(rc * RC + rg * 16, 16)]
          for rr in range(16):
            d2s = _lane_splat(d2v, rr)
            scl = jnp.where(final, _lane_splat(sdv, rr),
                            jnp.ones((16,), jnp.float32))
            r = rg * 16 + rr
            for j in range(NV):
              sl = pl.ds(j * 16, 16)
              v = ((1.0 - ALPHA) * d2s * ubuf[r, sl]
                   + ALPHA * g_tile[rc * RC + r, sl])
              ubuf[r, sl] = v * scl
          return 0
        lax.fori_loop(0, RC // 16, urow, 0)

        @pl.when(jnp.logical_not(final))
        def _():
          pltpu.sync_copy(ubuf, w_sp.at[pl.ds(rbase + rc * RC, RC)])
          pltpu.sync_copy(ubuf, acc_sp.at[pl.ds(rbase + rc * RC, RC)])

        @pl.when(final)
        def _():
          for rg in range(RC // 16):
            r0 = rbase + rc * RC + rg * 16

            @pl.when(r0 < N)
            def _():
              pltpu.sync_copy(ubuf.at[pl.ds(rg * 16, 16)],
                              out_hbm.at[bidx, pl.ds(r0, 16), pl.ds(0, C)])
      plsc.subcore_barrier()
      return 0

    lax.fori_loop(0, K, step, 0)


def _appnp(h, src, dst, d_feat):
  nblk = d_feat // (NCORES * C)
  hsplit = h.reshape(N, NCORES * nblk, C).transpose(1, 0, 2)
  mesh = plsc.VectorSubcoreMesh(core_axis_name="c", subcore_axis_name="s")
  kern = pl.kernel(
      functools.partial(_appnp_body, nblk),
      out_type=jax.ShapeDtypeStruct((NCORES * nblk, N, C), jnp.float32),
      mesh=mesh,
      compiler_params=pltpu.CompilerParams(use_tc_tiling_on_sc=False),
      scratch_types=[
          pltpu.VMEM((NCHUNK, CH), jnp.int32),
          pltpu.VMEM((NCHUNK, CH), jnp.int32),
          pltpu.VMEM((NBUF, CH, C), jnp.float32),
          pltpu.VMEM((RC, C), jnp.float32),
          pltpu.VMEM((RPT, C), jnp.float32),
          pltpu.VMEM((RPT,), jnp.float32),
          pltpu.VMEM((RPT,), jnp.float32),
          pltpu.VMEM((RPT,), jnp.float32),
          pltpu.VMEM((RPT,), jnp.float32),
          pltpu.VMEM((CH,), jnp.float32),
          pltpu.VMEM_SHARED((NPAD, C), jnp.float32),
          pltpu.VMEM_SHARED((NPAD, C), jnp.float32),
          pltpu.VMEM_SHARED((NPAD,), jnp.float32),
          pltpu.SemaphoreType.DMA((NBUF,)),
          pltpu.SemaphoreType.DMA((NBUF,)),
          pltpu.SemaphoreType.DMA,
      ],
  )
  out = kern(hsplit, src, dst)
  return out.transpose(1, 0, 2).reshape(N, d_feat)


def kernel(x, edge_index, W1, b1, W2, b2):
  src = edge_index[0]
  dst = edge_index[1]
  h = _matmul(x, W1, b1)
  z1 = _appnp(h, src, dst, 128)
  h2 = _matmul(z1, W2, b2, relu=True)
  return _appnp(h2, src, dst, 64)


# speedup vs baseline: 33.4867x; 1.5894x over previous
import functools

import jax
import jax.numpy as jnp
from jax import lax
from jax.experimental import pallas as pl
from jax.experimental.pallas import tpu as pltpu
from jax.experimental.pallas import tpu_sc as plsc

N = 10000
E = 320000
K = 10
ALPHA = 0.1

NTILES = 16
NCORES = 2
EPT = E // NTILES
CH = 128
NFULL = EPT // CH
TAIL = EPT - NFULL * CH
NCHUNK = 160
RPT = 640
NPAD = RPT * NTILES
DUMMY = N
RC = 128
NRC = RPT // RC
C = 32
NV = C // 16
NBUF = 5
LEAD = NBUF - 2
NT = NCHUNK // NBUF


def _mm_kernel(x_ref, w_ref, b_ref, o_ref, *, relu):
  x = x_ref[...]
  if relu:
    x = jnp.maximum(x, 0.0)
  o_ref[...] = jnp.dot(x, w_ref[...], preferred_element_type=jnp.float32) + b_ref[...]


def _matmul(x, w, b, relu=False):
  n, d_in = x.shape
  d_out = w.shape[1]
  blk = 1000
  grid = n // blk
  return pl.pallas_call(
      functools.partial(_mm_kernel, relu=relu),
      grid=(grid,),
      in_specs=[
          pl.BlockSpec((blk, d_in), lambda i: (i, 0)),
          pl.BlockSpec((d_in, d_out), lambda i: (0, 0)),
          pl.BlockSpec((d_out,), lambda i: (0,)),
      ],
      out_specs=pl.BlockSpec((blk, d_out), lambda i: (i, 0)),
      out_shape=jax.ShapeDtypeStruct((n, d_out), jnp.float32),
  )(x, w, b)


def _lane_splat(vec, rr):
  return lax.gather(
      vec, jnp.full((16, 1), rr, jnp.int32),
      dimension_numbers=lax.GatherDimensionNumbers(
          offset_dims=(), collapsed_slice_dims=(0,), start_index_map=(0,)),
      slice_sizes=(1,),
      mode=lax.GatherScatterMode.PROMISE_IN_BOUNDS)


def _appnp_body(NBLK, h_hbm, src_hbm, dst_hbm, out_hbm,
                src2d, dst2d, rowbufs, ubuf, obuf, g_tile,
                degb, dinvb, d2b, sdegb, onesb,
                w_sp, acc_sp, deg_sp, gsems, ssems, lsem):
  cid = lax.axis_index("c")
  sid = lax.axis_index("s")
  ebase = sid * EPT
  rbase = sid * RPT

  def fire_gather(c, b):
    return pltpu.async_copy(w_sp.at[src2d.at[c]], rowbufs.at[b], gsems.at[b])

  def wait_gather(b):
    pltpu.make_async_copy(w_sp.at[src2d.at[0]], rowbufs.at[b],
                          gsems.at[b]).wait()

  def fire_scatter(c, b):
    return pltpu.async_copy(rowbufs.at[b], acc_sp.at[dst2d.at[c]],
                            ssems.at[b], add=True)

  def wait_scatter(b):
    pltpu.make_async_copy(rowbufs.at[b], acc_sp.at[dst2d.at[0]],
                          ssems.at[b]).wait()

  def fire_load(ci, _):
    pltpu.async_copy(src_hbm.at[pl.ds(ebase + ci * CH, CH)], src2d.at[ci], lsem)
    pltpu.async_copy(dst_hbm.at[pl.ds(ebase + ci * CH, CH)], dst2d.at[ci], lsem)
    return 0
  lax.fori_loop(0, NFULL, fire_load, 0)
  pltpu.async_copy(src_hbm.at[pl.ds(ebase + NFULL * CH, TAIL)],
                   src2d.at[NFULL, pl.ds(0, TAIL)], lsem)
  pltpu.async_copy(dst_hbm.at[pl.ds(ebase + NFULL * CH, TAIL)],
                   dst2d.at[NFULL, pl.ds(0, TAIL)], lsem)

  def drain_load(ci, _):
    pltpu.make_async_copy(src_hbm.at[pl.ds(ebase, CH)], src2d.at[0], lsem).wait()
    pltpu.make_async_copy(dst_hbm.at[pl.ds(ebase, CH)], dst2d.at[0], lsem).wait()
    return 0
  lax.fori_loop(0, NFULL, drain_load, 0)
  pltpu.make_async_copy(src_hbm.at[pl.ds(ebase, TAIL)],
                        src2d.at[0, pl.ds(0, TAIL)], lsem).wait()
  pltpu.make_async_copy(dst_hbm.at[pl.ds(ebase, TAIL)],
                        dst2d.at[0, pl.ds(0, TAIL)], lsem).wait()

  for j in range(TAIL // 16, CH // 16):
    src2d[NFULL, pl.ds(j * 16, 16)] = jnp.zeros((16,), jnp.int32)
    dst2d[NFULL, pl.ds(j * 16, 16)] = jnp.full((16,), DUMMY, jnp.int32)
  for ci in range(NFULL + 1, NCHUNK):
    for j in range(CH // 16):
      src2d[ci, pl.ds(j * 16, 16)] = jnp.zeros((16,), jnp.int32)
      dst2d[ci, pl.ds(j * 16, 16)] = jnp.full((16,), DUMMY, jnp.int32)

  for j in range(RPT // 16):
    degb[pl.ds(j * 16, 16)] = jnp.zeros((16,), jnp.float32)
  for j in range(CH // 16):
    onesb[pl.ds(j * 16, 16)] = jnp.ones((16,), jnp.float32)
  pltpu.sync_copy(degb, deg_sp.at[pl.ds(rbase, RPT)])
  plsc.subcore_barrier()

  def deg_step(ci, _):
    pltpu.sync_copy(onesb, deg_sp.at[dst2d.at[ci]], add=True)
    return 0
  lax.fori_loop(0, NCHUNK, deg_step, 0)
  plsc.subcore_barrier()

  pltpu.sync_copy(deg_sp.at[pl.ds(rbase, RPT)], degb)
  for j in range(RPT // 16):
    sl = pl.ds(j * 16, 16)
    x = degb[sl] + 1.0
    i32 = lax.bitcast_convert_type(x, jnp.int32)
    i32 = jnp.full((16,), 0x5F3759DF, jnp.int32) - lax.shift_right_logical(
        i32, jnp.full((16,), 1, jnp.int32))
    y = lax.bitcast_convert_type(i32, jnp.float32)
    for _ in range(3):
      y = y * (1.5 - 0.5 * x * y * y)
    dinvb[sl] = y
    d2b[sl] = y * y
    sdegb[sl] = 1.0 / y

  def pack_row(dst_ref, r, va, vb):
    dst_ref[r, pl.ds(0, C)] = plsc.pack(va, vb,
                                        format=plsc.PackFormat.INTERLEAVED)

  def unpack_row(src_ref, r):
    return plsc.unpack(src_ref[r, pl.ds(0, C)],
                       format=plsc.PackFormat.INTERLEAVED)

  for blk in range(NBLK):
    bidx = cid * NBLK + blk

    for rc in range(NRC):
      for rg in range(RC // 16):
        r0 = rbase + rc * RC + rg * 16

        @pl.when(r0 < N)
        def _():
          pltpu.sync_copy(h_hbm.at[bidx, pl.ds(r0, 16), pl.ds(0, C)],
                          g_tile.at[pl.ds(rc * RC + rg * 16, 16)])

        @pl.when(r0 >= N)
        def _():
          for rr in range(16):
            for j in range(NV):
              g_tile[rc * RC + rg * 16 + rr, pl.ds(j * 16, 16)] = (
                  jnp.zeros((16,), jnp.float32))

    def grow(rg, _):
      dv = dinvb[pl.ds(rg * 16, 16)]
      for rr in range(16):
        s = _lane_splat(dv, rr)
        r = rg * 16 + rr
        for j in range(NV):
          g_tile[r, pl.ds(j * 16, 16)] = g_tile[r, pl.ds(j * 16, 16)] * s
      return 0
    lax.fori_loop(0, RPT // 16, grow, 0)
    for rc in range(NRC):
      def ginit(r, _):
        gr = rc * RC + r
        pack_row(ubuf, r, g_tile[gr, pl.ds(0, 16)], g_tile[gr, pl.ds(16, 16)])
        return 0
      lax.fori_loop(0, RC, ginit, 0)
      pltpu.sync_copy(ubuf, w_sp.at[pl.ds(rbase + rc * RC, RC)])
      pltpu.sync_copy(ubuf, acc_sp.at[pl.ds(rbase + rc * RC, RC)])
    plsc.subcore_barrier()

    def step(k, _):
      final = k == K - 1

      for b in range(LEAD):
        fire_gather(jnp.int32(b), b)

      def edge_iter(t, _):
        for b in range(NBUF):
          c = t * NBUF + b
          wait_gather(b)
          fire_scatter(c, b)
          g = c + LEAD
          bg = (b + LEAD) % NBUF
          if b < NBUF - LEAD:
            @pl.when(t >= 1)
            def _():
              wait_scatter(bg)
            fire_gather(g, bg)
          else:
            @pl.when(t < NT - 1)
            def _():
              wait_scatter(bg)
              fire_gather(g, bg)
        return 0
      lax.fori_loop(0, NT, edge_iter, 0)
      for b in range(NBUF):
        wait_scatter(b)
      plsc.subcore_barrier()

      for rc in range(NRC):
        pltpu.sync_copy(acc_sp.at[pl.ds(rbase + rc * RC, RC)], ubuf)

        def urow(rg, _):
          d2v = d2b[pl.ds(rc * RC + rg * 16, 16)]
          sdv = sdegb[pl.ds(rc * RC + rg * 16, 16)]
          for rr in range(16):
            d2s = _lane_splat(d2v, rr)
            scl = jnp.where(final, _lane_splat(sdv, rr),
                            jnp.ones((16,), jnp.float32))
            r = rg * 16 + rr
            ta, tb = unpack_row(ubuf, r)
            gr = rc * RC + r
            va = ((1.0 - ALPHA) * d2s * ta
                  + ALPHA * g_tile[gr, pl.ds(0, 16)]) * scl
            vb = ((1.0 - ALPHA) * d2s * tb
                  + ALPHA * g_tile[gr, pl.ds(16, 16)]) * scl
            pack_row(ubuf, r, va, vb)
            obuf[r, pl.ds(0, 16)] = va
            obuf[r, pl.ds(16, 16)] = vb
          return 0
        lax.fori_loop(0, RC // 16, urow, 0)

        @pl.when(jnp.logical_not(final))
        def _():
          pltpu.sync_copy(ubuf, w_sp.at[pl.ds(rbase + rc * RC, RC)])
          pltpu.sync_copy(ubuf, acc_sp.at[pl.ds(rbase + rc * RC, RC)])

        @pl.when(final)
        def _():
          for rg in range(RC // 16):
            r0 = rbase + rc * RC + rg * 16

            @pl.when(r0 < N)
            def _():
              pltpu.sync_copy(obuf.at[pl.ds(rg * 16, 16)],
                              out_hbm.at[bidx, pl.ds(r0, 16), pl.ds(0, C)])
      plsc.subcore_barrier()
      return 0

    lax.fori_loop(0, K, step, 0)


def _appnp(h, src, dst, d_feat):
  nblk = d_feat // (NCORES * C)
  hsplit = h.reshape(N, NCORES * nblk, C).transpose(1, 0, 2)
  mesh = plsc.VectorSubcoreMesh(core_axis_name="c", subcore_axis_name="s")
  kern = pl.kernel(
      functools.partial(_appnp_body, nblk),
      out_type=jax.ShapeDtypeStruct((NCORES * nblk, N, C), jnp.float32),
      mesh=mesh,
      compiler_params=pltpu.CompilerParams(use_tc_tiling_on_sc=False,
                                           needs_layout_passes=False),
      scratch_types=[
          pltpu.VMEM((NCHUNK, CH), jnp.int32),
          pltpu.VMEM((NCHUNK, CH), jnp.int32),
          pltpu.VMEM((NBUF, CH, C), jnp.bfloat16),
          pltpu.VMEM((RC, C), jnp.bfloat16),
          pltpu.VMEM((RC, C), jnp.float32),
          pltpu.VMEM((RPT, C), jnp.float32),
          pltpu.VMEM((RPT,), jnp.float32),
          pltpu.VMEM((RPT,), jnp.float32),
          pltpu.VMEM((RPT,), jnp.float32),
          pltpu.VMEM((RPT,), jnp.float32),
          pltpu.VMEM((CH,), jnp.float32),
          pltpu.VMEM_SHARED((NPAD, C), jnp.bfloat16),
          pltpu.VMEM_SHARED((NPAD, C), jnp.bfloat16),
          pltpu.VMEM_SHARED((NPAD,), jnp.float32),
          pltpu.SemaphoreType.DMA((NBUF,)),
          pltpu.SemaphoreType.DMA((NBUF,)),
          pltpu.SemaphoreType.DMA,
      ],
  )
  out = kern(hsplit, src, dst)
  return out.transpose(1, 0, 2).reshape(N, d_feat)


def kernel(x, edge_index, W1, b1, W2, b2):
  src = edge_index[0]
  dst = edge_index[1]
  h = _matmul(x, W1, b1)
  z1 = _appnp(h, src, dst, 128)
  h2 = _matmul(z1, W2, b2, relu=True)
  return _appnp(h2, src, dst, 64)


# speedup vs baseline: 33.5549x; 1.0020x over previous
import functools

import jax
import jax.numpy as jnp
from jax import lax
from jax.experimental import pallas as pl
from jax.experimental.pallas import tpu as pltpu
from jax.experimental.pallas import tpu_sc as plsc

N = 10000
E = 320000
K = 10
ALPHA = 0.1

NTILES = 16
NCORES = 2
EPT = E // NTILES
CH = 128
NFULL = EPT // CH
TAIL = EPT - NFULL * CH
NCHUNK = 160
RPT = 640
NPAD = RPT * NTILES
DUMMY = N
RC = 128
NRC = RPT // RC
C = 32
NV = C // 16
NBUF = 8
LEAD = NBUF - 2
NT = NCHUNK // NBUF


def _mm_kernel(x_ref, w_ref, b_ref, o_ref, *, relu):
  x = x_ref[...]
  if relu:
    x = jnp.maximum(x, 0.0)
  o_ref[...] = jnp.dot(x, w_ref[...], preferred_element_type=jnp.float32) + b_ref[...]


def _matmul(x, w, b, relu=False):
  n, d_in = x.shape
  d_out = w.shape[1]
  blk = 1000
  grid = n // blk
  return pl.pallas_call(
      functools.partial(_mm_kernel, relu=relu),
      grid=(grid,),
      in_specs=[
          pl.BlockSpec((blk, d_in), lambda i: (i, 0)),
          pl.BlockSpec((d_in, d_out), lambda i: (0, 0)),
          pl.BlockSpec((d_out,), lambda i: (0,)),
      ],
      out_specs=pl.BlockSpec((blk, d_out), lambda i: (i, 0)),
      out_shape=jax.ShapeDtypeStruct((n, d_out), jnp.float32),
  )(x, w, b)


def _lane_splat(vec, rr):
  return lax.gather(
      vec, jnp.full((16, 1), rr, jnp.int32),
      dimension_numbers=lax.GatherDimensionNumbers(
          offset_dims=(), collapsed_slice_dims=(0,), start_index_map=(0,)),
      slice_sizes=(1,),
      mode=lax.GatherScatterMode.PROMISE_IN_BOUNDS)


def _appnp_body(NBLK, h_hbm, src_hbm, dst_hbm, out_hbm,
                src2d, dst2d, rowbufs, ubuf, obuf, g_tile,
                degb, dinvb, d2b, sdegb, onesb,
                w_sp, acc_sp, deg_sp, gsems, ssems, lsem):
  cid = lax.axis_index("c")
  sid = lax.axis_index("s")
  ebase = sid * EPT
  rbase = sid * RPT

  def fire_gather(c, b):
    return pltpu.async_copy(w_sp.at[src2d.at[c]], rowbufs.at[b], gsems.at[b])

  def wait_gather(b):
    pltpu.make_async_copy(w_sp.at[src2d.at[0]], rowbufs.at[b],
                          gsems.at[b]).wait()

  def fire_scatter(c, b):
    return pltpu.async_copy(rowbufs.at[b], acc_sp.at[dst2d.at[c]],
                            ssems.at[b], add=True)

  def wait_scatter(b):
    pltpu.make_async_copy(rowbufs.at[b], acc_sp.at[dst2d.at[0]],
                          ssems.at[b]).wait()

  def fire_load(ci, _):
    pltpu.async_copy(src_hbm.at[pl.ds(ebase + ci * CH, CH)], src2d.at[ci], lsem)
    pltpu.async_copy(dst_hbm.at[pl.ds(ebase + ci * CH, CH)], dst2d.at[ci], lsem)
    return 0
  lax.fori_loop(0, NFULL, fire_load, 0)
  pltpu.async_copy(src_hbm.at[pl.ds(ebase + NFULL * CH, TAIL)],
                   src2d.at[NFULL, pl.ds(0, TAIL)], lsem)
  pltpu.async_copy(dst_hbm.at[pl.ds(ebase + NFULL * CH, TAIL)],
                   dst2d.at[NFULL, pl.ds(0, TAIL)], lsem)

  def drain_load(ci, _):
    pltpu.make_async_copy(src_hbm.at[pl.ds(ebase, CH)], src2d.at[0], lsem).wait()
    pltpu.make_async_copy(dst_hbm.at[pl.ds(ebase, CH)], dst2d.at[0], lsem).wait()
    return 0
  lax.fori_loop(0, NFULL, drain_load, 0)
  pltpu.make_async_copy(src_hbm.at[pl.ds(ebase, TAIL)],
                        src2d.at[0, pl.ds(0, TAIL)], lsem).wait()
  pltpu.make_async_copy(dst_hbm.at[pl.ds(ebase, TAIL)],
                        dst2d.at[0, pl.ds(0, TAIL)], lsem).wait()

  for j in range(TAIL // 16, CH // 16):
    src2d[NFULL, pl.ds(j * 16, 16)] = jnp.zeros((16,), jnp.int32)
    dst2d[NFULL, pl.ds(j * 16, 16)] = jnp.full((16,), DUMMY, jnp.int32)
  for ci in range(NFULL + 1, NCHUNK):
    for j in range(CH // 16):
      src2d[ci, pl.ds(j * 16, 16)] = jnp.zeros((16,), jnp.int32)
      dst2d[ci, pl.ds(j * 16, 16)] = jnp.full((16,), DUMMY, jnp.int32)

  for j in range(RPT // 16):
    degb[pl.ds(j * 16, 16)] = jnp.zeros((16,), jnp.float32)
  for j in range(CH // 16):
    onesb[pl.ds(j * 16, 16)] = jnp.ones((16,), jnp.float32)
  pltpu.sync_copy(degb, deg_sp.at[pl.ds(rbase, RPT)])
  plsc.subcore_barrier()

  def deg_step(ci, _):
    pltpu.sync_copy(onesb, deg_sp.at[dst2d.at[ci]], add=True)
    return 0
  lax.fori_loop(0, NCHUNK, deg_step, 0)
  plsc.subcore_barrier()

  pltpu.sync_copy(deg_sp.at[pl.ds(rbase, RPT)], degb)
  for j in range(RPT // 16):
    sl = pl.ds(j * 16, 16)
    x = degb[sl] + 1.0
    i32 = lax.bitcast_convert_type(x, jnp.int32)
    i32 = jnp.full((16,), 0x5F3759DF, jnp.int32) - lax.shift_right_logical(
        i32, jnp.full((16,), 1, jnp.int32))
    y = lax.bitcast_convert_type(i32, jnp.float32)
    for _ in range(3):
      y = y * (1.5 - 0.5 * x * y * y)
    dinvb[sl] = y
    d2b[sl] = y * y
    sdegb[sl] = 1.0 / y

  def pack_row(dst_ref, r, va, vb):
    dst_ref[r, pl.ds(0, C)] = plsc.pack(va, vb,
                                        format=plsc.PackFormat.INTERLEAVED)

  def unpack_row(src_ref, r):
    return plsc.unpack(src_ref[r, pl.ds(0, C)],
                       format=plsc.PackFormat.INTERLEAVED)

  for blk in range(NBLK):
    bidx = cid * NBLK + blk

    for rc in range(NRC):
      for rg in range(RC // 16):
        r0 = rbase + rc * RC + rg * 16

        @pl.when(r0 < N)
        def _():
          pltpu.sync_copy(h_hbm.at[bidx, pl.ds(r0, 16), pl.ds(0, C)],
                          g_tile.at[pl.ds(rc * RC + rg * 16, 16)])

        @pl.when(r0 >= N)
        def _():
          for rr in range(16):
            for j in range(NV):
              g_tile[rc * RC + rg * 16 + rr, pl.ds(j * 16, 16)] = (
                  jnp.zeros((16,), jnp.float32))

    def grow(rg, _):
      dv = dinvb[pl.ds(rg * 16, 16)]
      for rr in range(16):
        s = _lane_splat(dv, rr)
        r = rg * 16 + rr
        for j in range(NV):
          g_tile[r, pl.ds(j * 16, 16)] = g_tile[r, pl.ds(j * 16, 16)] * s
      return 0
    lax.fori_loop(0, RPT // 16, grow, 0)
    for rc in range(NRC):
      def ginit(r, _):
        gr = rc * RC + r
        pack_row(ubuf, r, g_tile[gr, pl.ds(0, 16)], g_tile[gr, pl.ds(16, 16)])
        return 0
      lax.fori_loop(0, RC, ginit, 0)
      pltpu.sync_copy(ubuf, w_sp.at[pl.ds(rbase + rc * RC, RC)])
      pltpu.sync_copy(ubuf, acc_sp.at[pl.ds(rbase + rc * RC, RC)])
    plsc.subcore_barrier()

    def step(k, _):
      final = k == K - 1

      for b in range(LEAD):
        fire_gather(jnp.int32(b), b)

      def edge_iter(t, _):
        for b in range(NBUF):
          c = t * NBUF + b
          wait_gather(b)
          fire_scatter(c, b)
          g = c + LEAD
          bg = (b + LEAD) % NBUF
          if b < NBUF - LEAD:
            @pl.when(t >= 1)
            def _():
              wait_scatter(bg)
            fire_gather(g, bg)
          else:
            @pl.when(t < NT - 1)
            def _():
              wait_scatter(bg)
              fire_gather(g, bg)
        return 0
      lax.fori_loop(0, NT, edge_iter, 0)
      for b in range(NBUF):
        wait_scatter(b)
      plsc.subcore_barrier()

      for rc in range(NRC):
        pltpu.sync_copy(acc_sp.at[pl.ds(rbase + rc * RC, RC)], ubuf)

        def urow(rg, _):
          d2v = d2b[pl.ds(rc * RC + rg * 16, 16)]
          sdv = sdegb[pl.ds(rc * RC + rg * 16, 16)]
          for rr in range(16):
            d2s = _lane_splat(d2v, rr)
            scl = jnp.where(final, _lane_splat(sdv, rr),
                            jnp.ones((16,), jnp.float32))
            r = rg * 16 + rr
            ta, tb = unpack_row(ubuf, r)
            gr = rc * RC + r
            va = ((1.0 - ALPHA) * d2s * ta
                  + ALPHA * g_tile[gr, pl.ds(0, 16)]) * scl
            vb = ((1.0 - ALPHA) * d2s * tb
                  + ALPHA * g_tile[gr, pl.ds(16, 16)]) * scl
            pack_row(ubuf, r, va, vb)
            obuf[r, pl.ds(0, 16)] = va
            obuf[r, pl.ds(16, 16)] = vb
          return 0
        lax.fori_loop(0, RC // 16, urow, 0)

        @pl.when(jnp.logical_not(final))
        def _():
          pltpu.sync_copy(ubuf, w_sp.at[pl.ds(rbase + rc * RC, RC)])
          pltpu.sync_copy(ubuf, acc_sp.at[pl.ds(rbase + rc * RC, RC)])

        @pl.when(final)
        def _():
          for rg in range(RC // 16):
            r0 = rbase + rc * RC + rg * 16

            @pl.when(r0 < N)
            def _():
              pltpu.sync_copy(obuf.at[pl.ds(rg * 16, 16)],
                              out_hbm.at[bidx, pl.ds(r0, 16), pl.ds(0, C)])
      plsc.subcore_barrier()
      return 0

    lax.fori_loop(0, K, step, 0)


def _appnp(h, src, dst, d_feat):
  nblk = d_feat // (NCORES * C)
  hsplit = h.reshape(N, NCORES * nblk, C).transpose(1, 0, 2)
  mesh = plsc.VectorSubcoreMesh(core_axis_name="c", subcore_axis_name="s")
  kern = pl.kernel(
      functools.partial(_appnp_body, nblk),
      out_type=jax.ShapeDtypeStruct((NCORES * nblk, N, C), jnp.float32),
      mesh=mesh,
      compiler_params=pltpu.CompilerParams(use_tc_tiling_on_sc=False,
                                           needs_layout_passes=False),
      scratch_types=[
          pltpu.VMEM((NCHUNK, CH), jnp.int32),
          pltpu.VMEM((NCHUNK, CH), jnp.int32),
          pltpu.VMEM((NBUF, CH, C), jnp.bfloat16),
          pltpu.VMEM((RC, C), jnp.bfloat16),
          pltpu.VMEM((RC, C), jnp.float32),
          pltpu.VMEM((RPT, C), jnp.float32),
          pltpu.VMEM((RPT,), jnp.float32),
          pltpu.VMEM((RPT,), jnp.float32),
          pltpu.VMEM((RPT,), jnp.float32),
          pltpu.VMEM((RPT,), jnp.float32),
          pltpu.VMEM((CH,), jnp.float32),
          pltpu.VMEM_SHARED((NPAD, C), jnp.bfloat16),
          pltpu.VMEM_SHARED((NPAD, C), jnp.bfloat16),
          pltpu.VMEM_SHARED((NPAD,), jnp.float32),
          pltpu.SemaphoreType.DMA((NBUF,)),
          pltpu.SemaphoreType.DMA((NBUF,)),
          pltpu.SemaphoreType.DMA,
      ],
  )
  out = kern(hsplit, src, dst)
  return out.transpose(1, 0, 2).reshape(N, d_feat)


def kernel(x, edge_index, W1, b1, W2, b2):
  src = edge_index[0]
  dst = edge_index[1]
  h = _matmul(x, W1, b1)
  z1 = _appnp(h, src, dst, 128)
  h2 = _matmul(z1, W2, b2, relu=True)
  return _appnp(h2, src, dst, 64)
